# Initial kernel scaffold; baseline (speedup 1.0000x reference)
import jax
import jax.numpy as jnp
from jax.experimental import pallas as pl


def kernel(x, edge_index, W_self1, W_neigh1, b1, W_self2, W_neigh2, b2):
    raise NotImplementedError("write your pallas kernel here")


# speedup vs baseline: 3.4097x; 3.4097x over previous
import functools

import jax
import jax.numpy as jnp
from jax import lax
from jax.experimental import pallas as pl
from jax.experimental.pallas import tpu as pltpu
from jax.experimental.pallas import tpu_sc as plsc

N = 10000
E = 320000
D = 128
DH = D // 2

NC = 2
NS = 16
CH = 80
NCH = 256
EPAD = NS * NCH * CH
NPAD = 12800
RPT = NPAD // NS
ZR = 80


def _seg_sum_body(with_deg, *refs):
    if with_deg:
        (featL, featR, src2, dst2, zeros2d, zeros1, ones_h,
         accout, degout, src_buf, dst_buf, rows, ones_v, acc, deg) = refs
    else:
        (featL, featR, src2, dst2, zeros2d,
         accout, src_buf, dst_buf, rows, acc) = refs

    c = lax.axis_index("c")
    s = lax.axis_index("s")

    for z in range(RPT // ZR):
        pltpu.sync_copy(zeros2d, acc.at[pl.ds(s * RPT + z * ZR, ZR)])
    if with_deg:
        @pl.when(c == 0)
        def _():
            pltpu.sync_copy(zeros1, deg.at[pl.ds(s * RPT, RPT)])
            pltpu.sync_copy(ones_h, ones_v)
    pltpu.sync_copy(src2.at[pl.ds(s * NCH, NCH)], src_buf)
    pltpu.sync_copy(dst2.at[pl.ds(s * NCH, NCH)], dst_buf)
    plsc.subcore_barrier()

    def chunk(j, carry):
        @pl.when(c == 0)
        def _():
            pltpu.sync_copy(featL.at[src_buf.at[j]], rows)

        @pl.when(c == 1)
        def _():
            pltpu.sync_copy(featR.at[src_buf.at[j]], rows)

        pltpu.sync_copy(rows, acc.at[dst_buf.at[j]], add=True)
        if with_deg:
            @pl.when(c == 0)
            def _():
                pltpu.sync_copy(ones_v, deg.at[dst_buf.at[j]], add=True)
        return carry

    lax.fori_loop(0, NCH, chunk, 0)
    plsc.subcore_barrier()

    base = c * NPAD + s * RPT
    pltpu.sync_copy(acc.at[pl.ds(s * RPT, RPT)],
                    accout.at[pl.ds(base, RPT)])
    if with_deg:
        @pl.when(c == 0)
        def _():
            pltpu.sync_copy(deg.at[pl.ds(s * RPT, RPT)],
                            degout.at[pl.ds(s * RPT, RPT)])


def _seg_sum(featL, featR, src2, dst2, with_deg):
    mesh = plsc.VectorSubcoreMesh(core_axis_name="c", subcore_axis_name="s")
    if with_deg:
        out_type = (jax.ShapeDtypeStruct((2 * NPAD, DH), jnp.float32),
                    jax.ShapeDtypeStruct((NPAD, 8), jnp.float32))
        scratch = [pltpu.VMEM((NCH, CH), jnp.int32),
                   pltpu.VMEM((NCH, CH), jnp.int32),
                   pltpu.VMEM((CH, DH), jnp.float32),
                   pltpu.VMEM((CH, 8), jnp.float32),
                   pltpu.VMEM_SHARED((NPAD, DH), jnp.float32),
                   pltpu.VMEM_SHARED((NPAD, 8), jnp.float32)]
        extra = (jnp.zeros((RPT, 8), jnp.float32),
                 jnp.ones((CH, 8), jnp.float32))
    else:
        out_type = jax.ShapeDtypeStruct((2 * NPAD, DH), jnp.float32)
        scratch = [pltpu.VMEM((NCH, CH), jnp.int32),
                   pltpu.VMEM((NCH, CH), jnp.int32),
                   pltpu.VMEM((CH, DH), jnp.float32),
                   pltpu.VMEM_SHARED((NPAD, DH), jnp.float32)]
        extra = ()
    fn = pl.kernel(
        functools.partial(_seg_sum_body, with_deg),
        out_type=out_type,
        mesh=mesh,
        scratch_types=scratch,
        compiler_params=pltpu.CompilerParams(use_tc_tiling_on_sc=False),
    )
    return fn(featL, featR, src2, dst2, jnp.zeros((ZR, DH), jnp.float32),
              *extra)


NBLK = 400
GRID = N // NBLK
OFFB = NPAD // NBLK


def _layer1_body(x, accA, accB, deg, ws, wn, b, h1, r):
    dd = jnp.maximum(deg[...][:, 0:1], 1.0)
    rr = 1.0 / dd
    hn = jnp.concatenate([accA[...], accB[...]], axis=1) * rr
    h = (jnp.dot(x[...], ws[...], preferred_element_type=jnp.float32)
         + jnp.dot(hn, wn[...], preferred_element_type=jnp.float32)
         + b[...])
    h1[...] = jnp.maximum(h, 0.0)
    r[...] = rr


def _layer2_body(h1, accA, accB, r, ws, wn, b, out):
    hn = jnp.concatenate([accA[...], accB[...]], axis=1) * r[...]
    out[...] = (jnp.dot(h1[...], ws[...], preferred_element_type=jnp.float32)
                + jnp.dot(hn, wn[...], preferred_element_type=jnp.float32)
                + b[...])


def _feat_spec():
    return pl.BlockSpec((NBLK, D), lambda i: (i, 0))


def _acc_specs():
    return [pl.BlockSpec((NBLK, DH), lambda i: (i, 0)),
            pl.BlockSpec((NBLK, DH), lambda i: (i + OFFB, 0))]


def _w_spec(d_out):
    return pl.BlockSpec((D, d_out), lambda i: (0, 0))


def _b_spec(d_out):
    return pl.BlockSpec((1, d_out), lambda i: (0, 0))


def kernel(x, edge_index, W_self1, W_neigh1, b1, W_self2, W_neigh2, b2):
    ei = edge_index.astype(jnp.int32)
    pad = EPAD - E
    src2 = jnp.concatenate(
        [ei[0], jnp.zeros((pad,), jnp.int32)]).reshape(EPAD // CH, CH)
    dst2 = jnp.concatenate(
        [ei[1], jnp.full((pad,), NPAD - 1, jnp.int32)]).reshape(EPAD // CH, CH)

    acc1, deg = _seg_sum(x[:, :DH], x[:, DH:], src2, dst2, with_deg=True)

    h1, r = pl.pallas_call(
        _layer1_body,
        grid=(GRID,),
        in_specs=[_feat_spec(), *_acc_specs(),
                  pl.BlockSpec((NBLK, 8), lambda i: (i, 0)),
                  _w_spec(D), _w_spec(D), _b_spec(D)],
        out_specs=[_feat_spec(), pl.BlockSpec((NBLK, 1), lambda i: (i, 0))],
        out_shape=[jax.ShapeDtypeStruct((N, D), jnp.float32),
                   jax.ShapeDtypeStruct((N, 1), jnp.float32)],
    )(x, acc1, acc1, deg, W_self1, W_neigh1, b1.reshape(1, D))

    acc2 = _seg_sum(h1[:, :DH], h1[:, DH:], src2, dst2, with_deg=False)

    C = W_self2.shape[1]
    out = pl.pallas_call(
        _layer2_body,
        grid=(GRID,),
        in_specs=[_feat_spec(), *_acc_specs(),
                  pl.BlockSpec((NBLK, 1), lambda i: (i, 0)),
                  _w_spec(C), _w_spec(C), _b_spec(C)],
        out_specs=pl.BlockSpec((NBLK, C), lambda i: (i, 0)),
        out_shape=jax.ShapeDtypeStruct((N, C), jnp.float32),
    )(h1, acc2, acc2, r, W_self2, W_neigh2, b2.reshape(1, C))
    return out


# speedup vs baseline: 4.0956x; 1.2012x over previous
import functools

import jax
import jax.numpy as jnp
from jax import lax
from jax.experimental import pallas as pl
from jax.experimental.pallas import tpu as pltpu
from jax.experimental.pallas import tpu_sc as plsc

N = 10000
E = 320000
D = 128
DH = D // 2

NC = 2
NS = 16
CH = 80
NCH = 256
NBUF = 4
NSTEP = NCH // NBUF
EPAD = NS * NCH * CH
NPAD = 10240
RPT = NPAD // NS


def _seg_sum_body(with_deg, *refs):
    if with_deg:
        (featL, featR, src2, dst2, zeros2d, zeros1, ones_h,
         accout, degout, src_buf, dst_buf, ones_v,
         rows0, rows1, rows2, rows3,
         gs0, gs1, gs2, gs3, ss0, ss1, ss2, ss3, ds0, ds1, ds2, ds3,
         acc, deg) = refs
        dsem = [ds0, ds1, ds2, ds3]
    else:
        (featL, featR, src2, dst2, zeros2d,
         accout, src_buf, dst_buf,
         rows0, rows1, rows2, rows3,
         gs0, gs1, gs2, gs3, ss0, ss1, ss2, ss3,
         acc) = refs
    rows = [rows0, rows1, rows2, rows3]
    gsem = [gs0, gs1, gs2, gs3]
    ssem = [ss0, ss1, ss2, ss3]

    c = lax.axis_index("c")
    s = lax.axis_index("s")

    def start_gather(j, b):
        @pl.when(c == 0)
        def _():
            pltpu.async_copy(featL.at[src_buf.at[j]], rows[b], gsem[b])

        @pl.when(c == 1)
        def _():
            pltpu.async_copy(featR.at[src_buf.at[j]], rows[b], gsem[b])

    def wait_gather(b):
        pltpu.make_async_copy(featL.at[src_buf.at[0]], rows[b],
                              gsem[b]).wait()

    def start_scatter(j, b):
        pltpu.make_async_copy(rows[b], acc.at[dst_buf.at[j]],
                              ssem[b]).start(add=True)

    def wait_scatter(b):
        pltpu.make_async_copy(rows[b], acc.at[dst_buf.at[0]],
                              ssem[b]).wait()

    def start_deg(j, b):
        pltpu.make_async_copy(ones_v, deg.at[dst_buf.at[j]],
                              dsem[b]).start(add=True)

    def wait_deg(b):
        pltpu.make_async_copy(ones_v, deg.at[dst_buf.at[0]],
                              dsem[b]).wait()

    pltpu.sync_copy(zeros2d, acc.at[pl.ds(s * RPT, RPT)])
    if with_deg:
        pltpu.sync_copy(zeros1, deg.at[pl.ds(s * RPT, RPT)])
        pltpu.sync_copy(ones_h, ones_v)
    pltpu.sync_copy(src2.at[pl.ds(s * NCH, NCH)], src_buf)
    pltpu.sync_copy(dst2.at[pl.ds(s * NCH, NCH)], dst_buf)
    plsc.subcore_barrier()

    for b in range(NBUF):
        start_gather(b, b)
    for b in range(NBUF):
        wait_gather(b)
        start_scatter(b, b)
        if with_deg:
            start_deg(b, b)
    for b in range(NBUF):
        wait_scatter(b)
        start_gather(NBUF + b, b)

    def step(g, carry):
        for b in range(NBUF):
            j = g * NBUF + b
            wait_gather(b)
            start_scatter(j, b)
            if with_deg:
                wait_deg(b)
                start_deg(j, b)
        for b in range(NBUF):
            wait_scatter(b)
            start_gather((g + 1) * NBUF + b, b)
        return carry

    lax.fori_loop(1, NSTEP - 1, step, 0)
    for b in range(NBUF):
        j = (NSTEP - 1) * NBUF + b
        wait_gather(b)
        start_scatter(j, b)
        if with_deg:
            wait_deg(b)
            start_deg(j, b)
    for b in range(NBUF):
        wait_scatter(b)
        if with_deg:
            wait_deg(b)
    plsc.subcore_barrier()

    base = c * NPAD + s * RPT
    pltpu.sync_copy(acc.at[pl.ds(s * RPT, RPT)],
                    accout.at[pl.ds(base, RPT)])
    if with_deg:
        pltpu.sync_copy(deg.at[pl.ds(s * RPT, RPT)],
                        degout.at[pl.ds(base, RPT)])


def _seg_sum(featL, featR, src2, dst2, with_deg):
    mesh = plsc.VectorSubcoreMesh(core_axis_name="c", subcore_axis_name="s")
    rows_bufs = [pltpu.VMEM((CH, DH), jnp.float32) for _ in range(NBUF)]
    if with_deg:
        out_type = (jax.ShapeDtypeStruct((2 * NPAD, DH), jnp.float32),
                    jax.ShapeDtypeStruct((2 * NPAD, 8), jnp.float32))
        scratch = ([pltpu.VMEM((NCH, CH), jnp.int32),
                    pltpu.VMEM((NCH, CH), jnp.int32),
                    pltpu.VMEM((CH, 8), jnp.float32)]
                   + rows_bufs
                   + [pltpu.SemaphoreType.DMA] * (3 * NBUF)
                   + [pltpu.VMEM_SHARED((NPAD, DH), jnp.float32),
                      pltpu.VMEM_SHARED((NPAD, 8), jnp.float32)])
        extra = (jnp.zeros((RPT, 8), jnp.float32),
                 jnp.ones((CH, 8), jnp.float32))
    else:
        out_type = jax.ShapeDtypeStruct((2 * NPAD, DH), jnp.float32)
        scratch = ([pltpu.VMEM((NCH, CH), jnp.int32),
                    pltpu.VMEM((NCH, CH), jnp.int32)]
                   + rows_bufs
                   + [pltpu.SemaphoreType.DMA] * (2 * NBUF)
                   + [pltpu.VMEM_SHARED((NPAD, DH), jnp.float32)])
        extra = ()
    fn = pl.kernel(
        functools.partial(_seg_sum_body, with_deg),
        out_type=out_type,
        mesh=mesh,
        scratch_types=scratch,
        compiler_params=pltpu.CompilerParams(use_tc_tiling_on_sc=False),
    )
    return fn(featL, featR, src2, dst2, jnp.zeros((RPT, DH), jnp.float32),
              *extra)


NBLK = 80
GRID = N // NBLK
OFFB = NPAD // NBLK


def _layer1_body(x, accA, accB, degA, degB, ws, wn, b, h1, r):
    dd = jnp.maximum(0.5 * (degA[...][:, 0:1] + degB[...][:, 0:1]), 1.0)
    rr = 1.0 / dd
    hn = jnp.concatenate([accA[...], accB[...]], axis=1) * rr
    h = (jnp.dot(x[...], ws[...], preferred_element_type=jnp.float32)
         + jnp.dot(hn, wn[...], preferred_element_type=jnp.float32)
         + b[...])
    h1[...] = jnp.maximum(h, 0.0)
    r[...] = rr


def _layer2_body(h1, accA, accB, r, ws, wn, b, out):
    hn = jnp.concatenate([accA[...], accB[...]], axis=1) * r[...]
    out[...] = (jnp.dot(h1[...], ws[...], preferred_element_type=jnp.float32)
                + jnp.dot(hn, wn[...], preferred_element_type=jnp.float32)
                + b[...])


def _feat_spec():
    return pl.BlockSpec((NBLK, D), lambda i: (i, 0))


def _acc_specs():
    return [pl.BlockSpec((NBLK, DH), lambda i: (i, 0)),
            pl.BlockSpec((NBLK, DH), lambda i: (i + OFFB, 0))]


def _w_spec(d_out):
    return pl.BlockSpec((D, d_out), lambda i: (0, 0))


def _b_spec(d_out):
    return pl.BlockSpec((1, d_out), lambda i: (0, 0))


def kernel(x, edge_index, W_self1, W_neigh1, b1, W_self2, W_neigh2, b2):
    ei = edge_index.astype(jnp.int32)
    pad = EPAD - E
    src2 = jnp.concatenate(
        [ei[0], jnp.zeros((pad,), jnp.int32)]).reshape(EPAD // CH, CH)
    dst2 = jnp.concatenate(
        [ei[1], jnp.full((pad,), NPAD - 1, jnp.int32)]).reshape(EPAD // CH, CH)

    acc1, deg = _seg_sum(x[:, :DH], x[:, DH:], src2, dst2, with_deg=True)

    h1, r = pl.pallas_call(
        _layer1_body,
        grid=(GRID,),
        in_specs=[_feat_spec(), *_acc_specs(),
                  pl.BlockSpec((NBLK, 8), lambda i: (i, 0)),
                  pl.BlockSpec((NBLK, 8), lambda i: (i + OFFB, 0)),
                  _w_spec(D), _w_spec(D), _b_spec(D)],
        out_specs=[_feat_spec(), pl.BlockSpec((NBLK, 1), lambda i: (i, 0))],
        out_shape=[jax.ShapeDtypeStruct((N, D), jnp.float32),
                   jax.ShapeDtypeStruct((N, 1), jnp.float32)],
    )(x, acc1, acc1, deg, deg, W_self1, W_neigh1, b1.reshape(1, D))

    acc2 = _seg_sum(h1[:, :DH], h1[:, DH:], src2, dst2, with_deg=False)

    C = W_self2.shape[1]
    out = pl.pallas_call(
        _layer2_body,
        grid=(GRID,),
        in_specs=[_feat_spec(), *_acc_specs(),
                  pl.BlockSpec((NBLK, 1), lambda i: (i, 0)),
                  _w_spec(C), _w_spec(C), _b_spec(C)],
        out_specs=pl.BlockSpec((NBLK, C), lambda i: (i, 0)),
        out_shape=jax.ShapeDtypeStruct((N, C), jnp.float32),
    )(h1, acc2, acc2, r, W_self2, W_neigh2, b2.reshape(1, C))
    return out


# speedup vs baseline: 4.1362x; 1.0099x over previous
import functools

import jax
import jax.numpy as jnp
from jax import lax
from jax.experimental import pallas as pl
from jax.experimental.pallas import tpu as pltpu
from jax.experimental.pallas import tpu_sc as plsc

N = 10000
E = 320000
D = 128
DH = D // 2

NC = 2
NS = 16
CH = 128
NCH = 160
NBUF = 4
NSTEP = NCH // NBUF
EPAD = NS * NCH * CH
NPAD = 10240
RPT = NPAD // NS


def _seg_sum_body(with_deg, *refs):
    if with_deg:
        (featL, featR, src2, dst2, zeros2d, zeros1, ones_h,
         accout, degout, src_buf, dst_buf, ones_v,
         rows0, rows1, rows2, rows3,
         gs0, gs1, gs2, gs3, ss0, ss1, ss2, ss3, ds0, ds1, ds2, ds3,
         acc, deg) = refs
        dsem = [ds0, ds1, ds2, ds3]
    else:
        (featL, featR, src2, dst2, zeros2d,
         accout, src_buf, dst_buf,
         rows0, rows1, rows2, rows3,
         gs0, gs1, gs2, gs3, ss0, ss1, ss2, ss3,
         acc) = refs
    rows = [rows0, rows1, rows2, rows3]
    gsem = [gs0, gs1, gs2, gs3]
    ssem = [ss0, ss1, ss2, ss3]

    c = lax.axis_index("c")
    s = lax.axis_index("s")

    def start_gather(j, b):
        @pl.when(c == 0)
        def _():
            pltpu.async_copy(featL.at[src_buf.at[j]], rows[b], gsem[b])

        @pl.when(c == 1)
        def _():
            pltpu.async_copy(featR.at[src_buf.at[j]], rows[b], gsem[b])

    def wait_gather(b):
        pltpu.make_async_copy(featL.at[src_buf.at[0]], rows[b],
                              gsem[b]).wait()

    def start_scatter(j, b):
        pltpu.make_async_copy(rows[b], acc.at[dst_buf.at[j]],
                              ssem[b]).start(add=True)

    def wait_scatter(b):
        pltpu.make_async_copy(rows[b], acc.at[dst_buf.at[0]],
                              ssem[b]).wait()

    def start_deg(j, b):
        pltpu.make_async_copy(ones_v, deg.at[dst_buf.at[j]],
                              dsem[b]).start(add=True)

    def wait_deg(b):
        pltpu.make_async_copy(ones_v, deg.at[dst_buf.at[0]],
                              dsem[b]).wait()

    pltpu.sync_copy(zeros2d, acc.at[pl.ds(s * RPT, RPT)])
    if with_deg:
        pltpu.sync_copy(zeros1, deg.at[pl.ds(s * RPT, RPT)])
        pltpu.sync_copy(ones_h, ones_v)
    pltpu.sync_copy(src2.at[pl.ds(s * NCH, NCH)], src_buf)
    pltpu.sync_copy(dst2.at[pl.ds(s * NCH, NCH)], dst_buf)
    plsc.subcore_barrier()

    for b in range(NBUF):
        start_gather(b, b)
    for b in range(NBUF):
        wait_gather(b)
        start_scatter(b, b)
        if with_deg:
            start_deg(b, b)
    for b in range(NBUF):
        wait_scatter(b)
        start_gather(NBUF + b, b)

    def step(g, carry):
        for b in range(NBUF):
            j = g * NBUF + b
            wait_gather(b)
            start_scatter(j, b)
            if with_deg:
                wait_deg(b)
                start_deg(j, b)
        for b in range(NBUF):
            wait_scatter(b)
            start_gather((g + 1) * NBUF + b, b)
        return carry

    lax.fori_loop(1, NSTEP - 1, step, 0)
    for b in range(NBUF):
        j = (NSTEP - 1) * NBUF + b
        wait_gather(b)
        start_scatter(j, b)
        if with_deg:
            wait_deg(b)
            start_deg(j, b)
    for b in range(NBUF):
        wait_scatter(b)
        if with_deg:
            wait_deg(b)
    plsc.subcore_barrier()

    base = c * NPAD + s * RPT
    pltpu.sync_copy(acc.at[pl.ds(s * RPT, RPT)],
                    accout.at[pl.ds(base, RPT)])
    if with_deg:
        pltpu.sync_copy(deg.at[pl.ds(s * RPT, RPT)],
                        degout.at[pl.ds(base, RPT)])


def _seg_sum(featL, featR, src2, dst2, with_deg):
    mesh = plsc.VectorSubcoreMesh(core_axis_name="c", subcore_axis_name="s")
    rows_bufs = [pltpu.VMEM((CH, DH), jnp.float32) for _ in range(NBUF)]
    if with_deg:
        out_type = (jax.ShapeDtypeStruct((2 * NPAD, DH), jnp.float32),
                    jax.ShapeDtypeStruct((2 * NPAD, 8), jnp.float32))
        scratch = ([pltpu.VMEM((NCH, CH), jnp.int32),
                    pltpu.VMEM((NCH, CH), jnp.int32),
                    pltpu.VMEM((CH, 8), jnp.float32)]
                   + rows_bufs
                   + [pltpu.SemaphoreType.DMA] * (3 * NBUF)
                   + [pltpu.VMEM_SHARED((NPAD, DH), jnp.float32),
                      pltpu.VMEM_SHARED((NPAD, 8), jnp.float32)])
        extra = (jnp.zeros((RPT, 8), jnp.float32),
                 jnp.ones((CH, 8), jnp.float32))
    else:
        out_type = jax.ShapeDtypeStruct((2 * NPAD, DH), jnp.float32)
        scratch = ([pltpu.VMEM((NCH, CH), jnp.int32),
                    pltpu.VMEM((NCH, CH), jnp.int32)]
                   + rows_bufs
                   + [pltpu.SemaphoreType.DMA] * (2 * NBUF)
                   + [pltpu.VMEM_SHARED((NPAD, DH), jnp.float32)])
        extra = ()
    fn = pl.kernel(
        functools.partial(_seg_sum_body, with_deg),
        out_type=out_type,
        mesh=mesh,
        scratch_types=scratch,
        compiler_params=pltpu.CompilerParams(use_tc_tiling_on_sc=False),
    )
    return fn(featL, featR, src2, dst2, jnp.zeros((RPT, DH), jnp.float32),
              *extra)


NBLK = 80
GRID = N // NBLK
OFFB = NPAD // NBLK


def _layer1_body(x, accA, accB, degA, degB, ws, wn, b, h1, r):
    dd = jnp.maximum(0.5 * (degA[...][:, 0:1] + degB[...][:, 0:1]), 1.0)
    rr = 1.0 / dd
    hn = jnp.concatenate([accA[...], accB[...]], axis=1) * rr
    h = (jnp.dot(x[...], ws[...], preferred_element_type=jnp.float32)
         + jnp.dot(hn, wn[...], preferred_element_type=jnp.float32)
         + b[...])
    h1[...] = jnp.maximum(h, 0.0)
    r[...] = rr


def _layer2_body(h1, accA, accB, r, ws, wn, b, out):
    hn = jnp.concatenate([accA[...], accB[...]], axis=1) * r[...]
    out[...] = (jnp.dot(h1[...], ws[...], preferred_element_type=jnp.float32)
                + jnp.dot(hn, wn[...], preferred_element_type=jnp.float32)
                + b[...])


def _feat_spec():
    return pl.BlockSpec((NBLK, D), lambda i: (i, 0))


def _acc_specs():
    return [pl.BlockSpec((NBLK, DH), lambda i: (i, 0)),
            pl.BlockSpec((NBLK, DH), lambda i: (i + OFFB, 0))]


def _w_spec(d_out):
    return pl.BlockSpec((D, d_out), lambda i: (0, 0))


def _b_spec(d_out):
    return pl.BlockSpec((1, d_out), lambda i: (0, 0))


def kernel(x, edge_index, W_self1, W_neigh1, b1, W_self2, W_neigh2, b2):
    ei = edge_index.astype(jnp.int32)
    pad = EPAD - E
    src2 = jnp.concatenate(
        [ei[0], jnp.zeros((pad,), jnp.int32)]).reshape(EPAD // CH, CH)
    dst2 = jnp.concatenate(
        [ei[1], jnp.full((pad,), NPAD - 1, jnp.int32)]).reshape(EPAD // CH, CH)

    acc1, deg = _seg_sum(x[:, :DH], x[:, DH:], src2, dst2, with_deg=True)

    h1, r = pl.pallas_call(
        _layer1_body,
        grid=(GRID,),
        in_specs=[_feat_spec(), *_acc_specs(),
                  pl.BlockSpec((NBLK, 8), lambda i: (i, 0)),
                  pl.BlockSpec((NBLK, 8), lambda i: (i + OFFB, 0)),
                  _w_spec(D), _w_spec(D), _b_spec(D)],
        out_specs=[_feat_spec(), pl.BlockSpec((NBLK, 1), lambda i: (i, 0))],
        out_shape=[jax.ShapeDtypeStruct((N, D), jnp.float32),
                   jax.ShapeDtypeStruct((N, 1), jnp.float32)],
    )(x, acc1, acc1, deg, deg, W_self1, W_neigh1, b1.reshape(1, D))

    acc2 = _seg_sum(h1[:, :DH], h1[:, DH:], src2, dst2, with_deg=False)

    C = W_self2.shape[1]
    out = pl.pallas_call(
        _layer2_body,
        grid=(GRID,),
        in_specs=[_feat_spec(), *_acc_specs(),
                  pl.BlockSpec((NBLK, 1), lambda i: (i, 0)),
                  _w_spec(C), _w_spec(C), _b_spec(C)],
        out_specs=pl.BlockSpec((NBLK, C), lambda i: (i, 0)),
        out_shape=jax.ShapeDtypeStruct((N, C), jnp.float32),
    )(h1, acc2, acc2, r, W_self2, W_neigh2, b2.reshape(1, C))
    return out


# speedup vs baseline: 4.1404x; 1.0010x over previous
import functools

import jax
import jax.numpy as jnp
from jax import lax
from jax.experimental import pallas as pl
from jax.experimental.pallas import tpu as pltpu
from jax.experimental.pallas import tpu_sc as plsc

N = 10000
E = 320000
D = 128
DH = D // 2

NC = 2
NS = 16
CH = 128
NCH = 160
NBUF = 4
NSTEP = NCH // NBUF
EPAD = NS * NCH * CH
NPAD = 10240
RPT = NPAD // NS


def _seg_sum_body(with_deg, *refs):
    if with_deg:
        (featL, featR, src2, dst2, zeros2d, zeros1, ones_h,
         accout, degout, src_buf, dst_buf, ones_v,
         rows0, rows1, rows2, rows3,
         gs0, gs1, gs2, gs3, ss0, ss1, ss2, ss3, ds0, ds1, ds2, ds3,
         acc, deg) = refs
        dsem = [ds0, ds1, ds2, ds3]
    else:
        (featL, featR, src2, dst2, zeros2d,
         accout, src_buf, dst_buf,
         rows0, rows1, rows2, rows3,
         gs0, gs1, gs2, gs3, ss0, ss1, ss2, ss3,
         acc) = refs
    rows = [rows0, rows1, rows2, rows3]
    gsem = [gs0, gs1, gs2, gs3]
    ssem = [ss0, ss1, ss2, ss3]

    c = lax.axis_index("c")
    s = lax.axis_index("s")

    def start_gather(j, b):
        @pl.when(c == 0)
        def _():
            pltpu.async_copy(featL.at[src_buf.at[j]], rows[b], gsem[b])

        @pl.when(c == 1)
        def _():
            pltpu.async_copy(featR.at[src_buf.at[j]], rows[b], gsem[b])

    def wait_gather(b):
        pltpu.make_async_copy(featL.at[src_buf.at[0]], rows[b],
                              gsem[b]).wait()

    def start_scatter(j, b):
        pltpu.make_async_copy(rows[b], acc.at[dst_buf.at[j]],
                              ssem[b]).start(add=True)

    def wait_scatter(b):
        pltpu.make_async_copy(rows[b], acc.at[dst_buf.at[0]],
                              ssem[b]).wait()

    def start_deg(j, b):
        pltpu.make_async_copy(ones_v, deg.at[dst_buf.at[j]],
                              dsem[b]).start(add=True)

    def wait_deg(b):
        pltpu.make_async_copy(ones_v, deg.at[dst_buf.at[0]],
                              dsem[b]).wait()

    pltpu.sync_copy(zeros2d, acc.at[pl.ds(s * RPT, RPT)])
    if with_deg:
        pltpu.sync_copy(zeros1, deg.at[pl.ds(s * RPT, RPT)])
        pltpu.sync_copy(ones_h, ones_v)
    pltpu.sync_copy(src2.at[pl.ds(s * NCH, NCH)], src_buf)
    pltpu.sync_copy(dst2.at[pl.ds(s * NCH, NCH)], dst_buf)
    plsc.subcore_barrier()

    for b in range(NBUF):
        start_gather(b, b)
    for b in range(NBUF):
        wait_gather(b)
        start_scatter(b, b)
        if with_deg:
            start_deg(b, b)
    for b in range(NBUF):
        wait_scatter(b)
        start_gather(NBUF + b, b)

    def step(g, carry):
        for b in range(NBUF):
            j = g * NBUF + b
            wait_gather(b)
            start_scatter(j, b)
            if with_deg:
                wait_deg(b)
                start_deg(j, b)
        for b in range(NBUF):
            wait_scatter(b)
            start_gather((g + 1) * NBUF + b, b)
        return carry

    lax.fori_loop(1, NSTEP - 1, step, 0)
    for b in range(NBUF):
        j = (NSTEP - 1) * NBUF + b
        wait_gather(b)
        start_scatter(j, b)
        if with_deg:
            wait_deg(b)
            start_deg(j, b)
    for b in range(NBUF):
        wait_scatter(b)
        if with_deg:
            wait_deg(b)
    plsc.subcore_barrier()

    base = c * NPAD + s * RPT
    pltpu.sync_copy(acc.at[pl.ds(s * RPT, RPT)],
                    accout.at[pl.ds(base, RPT)])
    if with_deg:
        pltpu.sync_copy(deg.at[pl.ds(s * RPT, RPT)],
                        degout.at[pl.ds(base, RPT)])


def _seg_sum(featL, featR, src2, dst2, with_deg):
    mesh = plsc.VectorSubcoreMesh(core_axis_name="c", subcore_axis_name="s")
    rows_bufs = [pltpu.VMEM((CH, DH), jnp.float32) for _ in range(NBUF)]
    if with_deg:
        out_type = (jax.ShapeDtypeStruct((2 * NPAD, DH), jnp.float32),
                    jax.ShapeDtypeStruct((2 * NPAD, 8), jnp.float32))
        scratch = ([pltpu.VMEM((NCH, CH), jnp.int32),
                    pltpu.VMEM((NCH, CH), jnp.int32),
                    pltpu.VMEM((CH, 8), jnp.float32)]
                   + rows_bufs
                   + [pltpu.SemaphoreType.DMA] * (3 * NBUF)
                   + [pltpu.VMEM_SHARED((NPAD, DH), jnp.float32),
                      pltpu.VMEM_SHARED((NPAD, 8), jnp.float32)])
        extra = (jnp.zeros((RPT, 8), jnp.float32),
                 jnp.ones((CH, 8), jnp.float32))
    else:
        out_type = jax.ShapeDtypeStruct((2 * NPAD, DH), jnp.float32)
        scratch = ([pltpu.VMEM((NCH, CH), jnp.int32),
                    pltpu.VMEM((NCH, CH), jnp.int32)]
                   + rows_bufs
                   + [pltpu.SemaphoreType.DMA] * (2 * NBUF)
                   + [pltpu.VMEM_SHARED((NPAD, DH), jnp.float32)])
        extra = ()
    fn = pl.kernel(
        functools.partial(_seg_sum_body, with_deg),
        out_type=out_type,
        mesh=mesh,
        scratch_types=scratch,
        compiler_params=pltpu.CompilerParams(use_tc_tiling_on_sc=False),
    )
    return fn(featL, featR, src2, dst2, jnp.zeros((RPT, DH), jnp.float32),
              *extra)


NBLK = 80
GRID = N // NBLK
OFFB = NPAD // NBLK


def _layer1_body(x, accA, accB, degA, degB, ws, wn, b, h1L, h1R, r):
    dd = jnp.maximum(0.5 * (degA[...][:, 0:1] + degB[...][:, 0:1]), 1.0)
    rr = 1.0 / dd
    hn = jnp.concatenate([accA[...], accB[...]], axis=1) * rr
    h = (jnp.dot(x[...], ws[...], preferred_element_type=jnp.float32)
         + jnp.dot(hn, wn[...], preferred_element_type=jnp.float32)
         + b[...])
    h = jnp.maximum(h, 0.0)
    h1L[...] = h[:, :DH]
    h1R[...] = h[:, DH:]
    r[...] = rr


def _layer2_body(h1L, h1R, accA, accB, r, ws, wn, b, out):
    hs = jnp.concatenate([h1L[...], h1R[...]], axis=1)
    hn = jnp.concatenate([accA[...], accB[...]], axis=1) * r[...]
    out[...] = (jnp.dot(hs, ws[...], preferred_element_type=jnp.float32)
                + jnp.dot(hn, wn[...], preferred_element_type=jnp.float32)
                + b[...])


def _feat_spec():
    return pl.BlockSpec((NBLK, D), lambda i: (i, 0))


def _acc_specs():
    return [pl.BlockSpec((NBLK, DH), lambda i: (i, 0)),
            pl.BlockSpec((NBLK, DH), lambda i: (i + OFFB, 0))]


def _w_spec(d_out):
    return pl.BlockSpec((D, d_out), lambda i: (0, 0))


def _b_spec(d_out):
    return pl.BlockSpec((1, d_out), lambda i: (0, 0))


def kernel(x, edge_index, W_self1, W_neigh1, b1, W_self2, W_neigh2, b2):
    ei = edge_index.astype(jnp.int32)
    pad = EPAD - E
    src2 = jnp.concatenate(
        [ei[0], jnp.zeros((pad,), jnp.int32)]).reshape(EPAD // CH, CH)
    dst2 = jnp.concatenate(
        [ei[1], jnp.full((pad,), NPAD - 1, jnp.int32)]).reshape(EPAD // CH, CH)

    acc1, deg = _seg_sum(x[:, :DH], x[:, DH:], src2, dst2, with_deg=True)

    h1L, h1R, r = pl.pallas_call(
        _layer1_body,
        grid=(GRID,),
        in_specs=[_feat_spec(), *_acc_specs(),
                  pl.BlockSpec((NBLK, 8), lambda i: (i, 0)),
                  pl.BlockSpec((NBLK, 8), lambda i: (i + OFFB, 0)),
                  _w_spec(D), _w_spec(D), _b_spec(D)],
        out_specs=[pl.BlockSpec((NBLK, DH), lambda i: (i, 0)),
                   pl.BlockSpec((NBLK, DH), lambda i: (i, 0)),
                   pl.BlockSpec((NBLK, 1), lambda i: (i, 0))],
        out_shape=[jax.ShapeDtypeStruct((N, DH), jnp.float32),
                   jax.ShapeDtypeStruct((N, DH), jnp.float32),
                   jax.ShapeDtypeStruct((N, 1), jnp.float32)],
    )(x, acc1, acc1, deg, deg, W_self1, W_neigh1, b1.reshape(1, D))

    acc2 = _seg_sum(h1L, h1R, src2, dst2, with_deg=False)

    C = W_self2.shape[1]
    out = pl.pallas_call(
        _layer2_body,
        grid=(GRID,),
        in_specs=[pl.BlockSpec((NBLK, DH), lambda i: (i, 0)),
                  pl.BlockSpec((NBLK, DH), lambda i: (i, 0)),
                  *_acc_specs(),
                  pl.BlockSpec((NBLK, 1), lambda i: (i, 0)),
                  _w_spec(C), _w_spec(C), _b_spec(C)],
        out_specs=pl.BlockSpec((NBLK, C), lambda i: (i, 0)),
        out_shape=jax.ShapeDtypeStruct((N, C), jnp.float32),
    )(h1L, h1R, acc2, acc2, r, W_self2, W_neigh2, b2.reshape(1, C))
    return out
